# Initial kernel scaffold; baseline (speedup 1.0000x reference)
import jax
import jax.numpy as jnp
from jax.experimental import pallas as pl


def kernel(raw_dense, raw_sparse, embedding_table, cls_w, cls_b):
    raise NotImplementedError("write your pallas kernel here")


# speedup vs baseline: 2.2558x; 2.2558x over previous
import functools

import numpy as np
import jax
import jax.numpy as jnp
from jax import lax
from jax.experimental import pallas as pl
from jax.experimental.pallas import tpu as pltpu
from jax.experimental.pallas import tpu_sc as plsc

NUM_DENSE_F = 13
NUM_SPARSE_F = 26
EMB_D = 16
VOCAB = 999999
BATCH_N = 4096
REG_COEF = 1e-05
SPARSE_W = NUM_SPARSE_F * EMB_D
_OFFSETS = np.array([0] + [13 + j * 38461 for j in range(1, NUM_SPARSE_F)],
                    dtype=np.int32)

NW = 32
BPW = BATCH_N // NW

_sc_mesh = plsc.VectorSubcoreMesh(core_axis_name="c", subcore_axis_name="s")


@functools.partial(
    pl.kernel,
    mesh=_sc_mesh,
    out_type=jax.ShapeDtypeStruct((BATCH_N,), jnp.float32),
    scratch_types=[
        pltpu.VMEM((NUM_SPARSE_F, BPW), jnp.int32),
        pltpu.VMEM((NUM_SPARSE_F * BPW, EMB_D), jnp.float32),
        pltpu.VMEM((NUM_SPARSE_F, EMB_D), jnp.float32),
        pltpu.VMEM((BPW,), jnp.float32),
        pltpu.SemaphoreType.DMA,
    ],
    compiler_params=pltpu.CompilerParams(
        needs_layout_passes=False, use_tc_tiling_on_sc=False),
)
def _sc_gather_dot(table_hbm, idx_hbm, ws_hbm, out_hbm,
                   idx_v, rows_v, ws_v, out_v, gsem):
    w = lax.axis_index("s") * 2 + lax.axis_index("c")
    pltpu.sync_copy(idx_hbm.at[w], idx_v)
    pltpu.sync_copy(ws_hbm, ws_v)
    gets = [
        pltpu.async_copy(table_hbm.at[idx_v.at[f]],
                         rows_v.at[pl.ds(f * BPW, BPW)], gsem)
        for f in range(NUM_SPARSE_F)
    ]
    for g in gets:
        g.wait()

    wvecs = [ws_v[f, :] for f in range(NUM_SPARSE_F)]
    lane = lax.iota(jnp.int32, 16)

    def body(g, _):
        vec = jnp.zeros((16,), jnp.float32)
        base = g * 16
        for j in range(16):
            i = base + j
            acc = rows_v[i, :] * wvecs[0]
            for f in range(1, NUM_SPARSE_F):
                acc = acc + rows_v[f * BPW + i, :] * wvecs[f]
            vec = jnp.where(lane == j, jnp.sum(acc), vec)
        out_v[pl.ds(base, 16)] = vec
        return 0

    lax.fori_loop(0, BPW // 16, body, 0)
    pltpu.sync_copy(out_v, out_hbm.at[pl.ds(w * BPW, BPW)])


RBLK = 8192
GRID_N = (VOCAB + RBLK - 1) // RBLK


def _tc_body(table_ref, s_ref, rd_ref, wd_ref, ws_ref, cb_ref,
             logits_ref, regs_ref, ssq_ref):
    pid = pl.program_id(0)

    @pl.when(pid == 0)
    def _init():
        ssq_ref[0] = 0.0
        tab0 = table_ref[...]
        d = jnp.sum(tab0[:NUM_DENSE_F, :] * wd_ref[...], axis=1)
        logits_d = jnp.sum(rd_ref[...] * d[None, :], axis=1, keepdims=True)
        logits_ref[...] = s_ref[...] + logits_d + cb_ref[...]

    @pl.when(pid < GRID_N - 1)
    def _acc():
        x = table_ref[...]
        ssq_ref[0] += jnp.sum(x * x)

    @pl.when(pid == GRID_N - 1)
    def _fin():
        x = table_ref[...]
        rowid = lax.broadcasted_iota(jnp.int32, (RBLK, EMB_D), 0)
        mask = (pid * RBLK + rowid) < VOCAB
        xm = jnp.where(mask, x, 0.0)
        ssq_ref[0] += jnp.sum(xm * xm)
        wd = wd_ref[...]
        ws = ws_ref[...]
        cw_ssq = jnp.sum(wd * wd) + jnp.sum(ws * ws)
        nb = jnp.abs(cb_ref[...])
        regs_ref[...] = REG_COEF * (
            jnp.sqrt(ssq_ref[0]) + jnp.sqrt(cw_ssq) + nb)


_tc_call = pl.pallas_call(
    _tc_body,
    grid=(GRID_N,),
    in_specs=[
        pl.BlockSpec((RBLK, EMB_D), lambda i: (i, 0)),
        pl.BlockSpec((BATCH_N, 1), lambda i: (0, 0)),
        pl.BlockSpec((BATCH_N, NUM_DENSE_F), lambda i: (0, 0)),
        pl.BlockSpec((NUM_DENSE_F, EMB_D), lambda i: (0, 0)),
        pl.BlockSpec((SPARSE_W, 1), lambda i: (0, 0)),
        pl.BlockSpec((1, 1), lambda i: (0, 0)),
    ],
    out_specs=[
        pl.BlockSpec((BATCH_N, 1), lambda i: (0, 0)),
        pl.BlockSpec((1, 1), lambda i: (0, 0)),
    ],
    out_shape=[
        jax.ShapeDtypeStruct((BATCH_N, 1), jnp.float32),
        jax.ShapeDtypeStruct((1, 1), jnp.float32),
    ],
    scratch_shapes=[pltpu.SMEM((1,), jnp.float32)],
)


def kernel(raw_dense, raw_sparse, embedding_table, cls_w, cls_b):
    idx = raw_sparse.astype(jnp.int32) + jnp.asarray(_OFFSETS)[None, :]
    idx_wfb = idx.reshape(NW, BPW, NUM_SPARSE_F).transpose(0, 2, 1)
    ws2 = cls_w[NUM_DENSE_F * EMB_D:, 0].reshape(NUM_SPARSE_F, EMB_D)
    s = _sc_gather_dot(embedding_table, idx_wfb, ws2)

    wd = cls_w[: NUM_DENSE_F * EMB_D, 0].reshape(NUM_DENSE_F, EMB_D)
    ws = cls_w[NUM_DENSE_F * EMB_D:, :]
    cb = cls_b.reshape(1, 1)
    logits, regs = _tc_call(embedding_table, s.reshape(BATCH_N, 1),
                            raw_dense, wd, ws, cb)
    return (logits, regs.reshape(()))


# speedup vs baseline: 2.9166x; 1.2929x over previous
import functools

import numpy as np
import jax
import jax.numpy as jnp
from jax import lax
from jax.experimental import pallas as pl
from jax.experimental.pallas import tpu as pltpu
from jax.experimental.pallas import tpu_sc as plsc

NUM_DENSE_F = 13
NUM_SPARSE_F = 26
EMB_D = 16
VOCAB = 999999
BATCH_N = 4096
REG_COEF = 1e-05
_OFFSETS = np.array([0] + [13 + j * 38461 for j in range(1, NUM_SPARSE_F)],
                    dtype=np.int32)

NW = 32
BPW = BATCH_N // NW
ROWS_PW = 31250
CHUNK = 1250
NCHUNK = ROWS_PW // CHUNK

_sc_mesh = plsc.VectorSubcoreMesh(core_axis_name="c", subcore_axis_name="s")


@functools.partial(
    pl.kernel,
    mesh=_sc_mesh,
    out_type=(
        jax.ShapeDtypeStruct((BATCH_N,), jnp.float32),
        jax.ShapeDtypeStruct((NW * 16,), jnp.float32),
    ),
    scratch_types=[
        pltpu.VMEM((NUM_SPARSE_F, BPW), jnp.int32),
        pltpu.VMEM((NUM_SPARSE_F * BPW, EMB_D), jnp.float32),
        pltpu.VMEM((NUM_SPARSE_F, EMB_D), jnp.float32),
        pltpu.VMEM((BPW,), jnp.float32),
        pltpu.VMEM((16,), jnp.float32),
        pltpu.VMEM((2, CHUNK, EMB_D), jnp.float32),
        pltpu.SemaphoreType.DMA,
        pltpu.SemaphoreType.DMA,
    ],
    compiler_params=pltpu.CompilerParams(
        needs_layout_passes=False, use_tc_tiling_on_sc=False),
)
def _sc_main(table_hbm, idx_hbm, ws_hbm, out_hbm, ssq_hbm,
             idx_v, rows_v, ws_v, out_v, ssq_v, bufs, gsem, nsem):
    w = lax.axis_index("s") * 2 + lax.axis_index("c")
    pltpu.sync_copy(idx_hbm.at[w], idx_v)
    pltpu.sync_copy(ws_hbm, ws_v)

    gets = [
        pltpu.async_copy(table_hbm.at[idx_v.at[f]],
                         rows_v.at[pl.ds(f * BPW, BPW)], gsem)
        for f in range(NUM_SPARSE_F)
    ]

    base = w * ROWS_PW

    def chunk_start(c):
        return jnp.minimum(base + c * CHUNK, VOCAB - CHUNK)

    dmas = [None] * NCHUNK
    dmas[0] = pltpu.async_copy(table_hbm.at[pl.ds(chunk_start(0), CHUNK)],
                               bufs.at[0], nsem)
    accs = tuple(jnp.zeros((16,), jnp.float32) for _ in range(5))

    def accum_chunk(buf, accs):
        def grp(i, accs):
            r = i * 10
            out = []
            for j in range(5):
                a = accs[j]
                v0 = buf[r + 2 * j, :]
                v1 = buf[r + 2 * j + 1, :]
                out.append(a + v0 * v0 + v1 * v1)
            return tuple(out)
        return lax.fori_loop(0, CHUNK // 10, grp, accs)

    for c in range(NCHUNK):
        if c + 1 < NCHUNK:
            dmas[c + 1] = pltpu.async_copy(
                table_hbm.at[pl.ds(chunk_start(c + 1), CHUNK)],
                bufs.at[(c + 1) % 2], nsem)
        dmas[c].wait()
        accs = accum_chunk(bufs.at[c % 2], accs)

    ssq_vec = accs[0] + accs[1] + accs[2] + accs[3] + accs[4]
    v_dup = bufs[(NCHUNK - 1) % 2, 0, :]
    ssq_vec = jnp.where(w == NW - 1, ssq_vec - v_dup * v_dup, ssq_vec)
    ssq_v[...] = ssq_vec
    pltpu.sync_copy(ssq_v, ssq_hbm.at[pl.ds(w * 16, 16)])

    for g in gets:
        g.wait()

    wvecs = [ws_v[f, :] for f in range(NUM_SPARSE_F)]
    lane = lax.iota(jnp.int32, 16)

    def body(g, _):
        vec = jnp.zeros((16,), jnp.float32)
        gbase = g * 16
        for j in range(16):
            i = gbase + j
            acc = rows_v[i, :] * wvecs[0]
            for f in range(1, NUM_SPARSE_F):
                acc = acc + rows_v[f * BPW + i, :] * wvecs[f]
            vec = jnp.where(lane == j, jnp.sum(acc), vec)
        out_v[pl.ds(gbase, 16)] = vec
        return 0

    lax.fori_loop(0, BPW // 16, body, 0)
    pltpu.sync_copy(out_v, out_hbm.at[pl.ds(w * BPW, BPW)])


def _tc_body(table_ref, s_ref, rd_ref, wd_ref, ws_ref, cb_ref, ssqv_ref,
             logits_ref, regs_ref):
    tab0 = table_ref[...]
    d = jnp.sum(tab0[:NUM_DENSE_F, :] * wd_ref[...], axis=1)
    logits_d = jnp.sum(rd_ref[...] * d[None, :], axis=1, keepdims=True)
    logits_ref[...] = s_ref[...] + logits_d + cb_ref[...]

    wd = wd_ref[...]
    ws = ws_ref[...]
    cw_ssq = jnp.sum(wd * wd) + jnp.sum(ws * ws)
    t_ssq = jnp.sum(ssqv_ref[...])
    nb = jnp.abs(cb_ref[...])
    regs_ref[...] = REG_COEF * (jnp.sqrt(t_ssq) + jnp.sqrt(cw_ssq) + nb)


_tc_call = pl.pallas_call(
    _tc_body,
    grid=(1,),
    in_specs=[
        pl.BlockSpec((16, EMB_D), lambda i: (0, 0)),
        pl.BlockSpec((BATCH_N, 1), lambda i: (0, 0)),
        pl.BlockSpec((BATCH_N, NUM_DENSE_F), lambda i: (0, 0)),
        pl.BlockSpec((NUM_DENSE_F, EMB_D), lambda i: (0, 0)),
        pl.BlockSpec((NUM_SPARSE_F * EMB_D, 1), lambda i: (0, 0)),
        pl.BlockSpec((1, 1), lambda i: (0, 0)),
        pl.BlockSpec((4, 128), lambda i: (0, 0)),
    ],
    out_specs=[
        pl.BlockSpec((BATCH_N, 1), lambda i: (0, 0)),
        pl.BlockSpec((1, 1), lambda i: (0, 0)),
    ],
    out_shape=[
        jax.ShapeDtypeStruct((BATCH_N, 1), jnp.float32),
        jax.ShapeDtypeStruct((1, 1), jnp.float32),
    ],
)


def kernel(raw_dense, raw_sparse, embedding_table, cls_w, cls_b):
    idx = raw_sparse.astype(jnp.int32) + jnp.asarray(_OFFSETS)[None, :]
    idx_wfb = idx.reshape(NW, BPW, NUM_SPARSE_F).transpose(0, 2, 1)
    ws2 = cls_w[NUM_DENSE_F * EMB_D:, 0].reshape(NUM_SPARSE_F, EMB_D)
    s, ssqv = _sc_main(embedding_table, idx_wfb, ws2)

    wd = cls_w[: NUM_DENSE_F * EMB_D, 0].reshape(NUM_DENSE_F, EMB_D)
    ws = cls_w[NUM_DENSE_F * EMB_D:, :]
    cb = cls_b.reshape(1, 1)
    logits, regs = _tc_call(embedding_table, s.reshape(BATCH_N, 1),
                            raw_dense, wd, ws, cb, ssqv.reshape(4, 128))
    return (logits, regs.reshape(()))


# speedup vs baseline: 3.6464x; 1.2502x over previous
import functools

import numpy as np
import jax
import jax.numpy as jnp
from jax import lax
from jax.experimental import pallas as pl
from jax.experimental.pallas import tpu as pltpu
from jax.experimental.pallas import tpu_sc as plsc

NUM_DENSE_F = 13
NUM_SPARSE_F = 26
EMB_D = 16
VOCAB = 999999
BATCH_N = 4096
REG_COEF = 1e-05
_OFFSETS = [0] + [13 + j * 38461 for j in range(1, NUM_SPARSE_F)]

NW = 32
BPW = BATCH_N // NW
ROWS_PW = 31250
CHUNK = 1250
NCHUNK = ROWS_PW // CHUNK
CW_ROWS = (NUM_DENSE_F + NUM_SPARSE_F)

_sc_mesh = plsc.VectorSubcoreMesh(core_axis_name="c", subcore_axis_name="s")


@functools.partial(
    pl.kernel,
    mesh=_sc_mesh,
    out_type=(
        jax.ShapeDtypeStruct((BATCH_N,), jnp.float32),
        jax.ShapeDtypeStruct((48, 16), jnp.float32),
    ),
    scratch_types=[
        pltpu.VMEM((BPW, NUM_SPARSE_F), jnp.int32),
        pltpu.VMEM((NUM_SPARSE_F * BPW,), jnp.int32),
        pltpu.VMEM((NUM_SPARSE_F * BPW, EMB_D), jnp.float32),
        pltpu.VMEM((CW_ROWS, EMB_D), jnp.float32),
        pltpu.VMEM((BPW, EMB_D), jnp.float32),
        pltpu.VMEM((16, EMB_D), jnp.float32),
        pltpu.VMEM((BPW,), jnp.float32),
        pltpu.VMEM((16,), jnp.float32),
        pltpu.VMEM((2, CHUNK, EMB_D), jnp.float32),
        pltpu.SemaphoreType.DMA,
        pltpu.SemaphoreType.DMA,
        pltpu.SemaphoreType.DMA,
    ],
    compiler_params=pltpu.CompilerParams(
        needs_layout_passes=False, use_tc_tiling_on_sc=False),
)
def _sc_main(table_hbm, rsp_hbm, cw_hbm, rd_hbm, out_hbm, ssq_hbm,
             idxb_v, idx_v, rows_v, cw_v, rd_v, tab0_v, out_v, ssq_v, bufs,
             gsem, nsem, psem):
    w = lax.axis_index("s") * 2 + lax.axis_index("c")
    ins = [
        pltpu.async_copy(rsp_hbm.at[pl.ds(w * BPW, BPW)], idxb_v, psem),
        pltpu.async_copy(cw_hbm, cw_v, psem),
        pltpu.async_copy(rd_hbm.at[pl.ds(w * BPW, BPW)], rd_v, psem),
        pltpu.async_copy(table_hbm.at[pl.ds(0, 16)], tab0_v, psem),
    ]
    for i in ins:
        i.wait()

    lane = lax.iota(jnp.int32, 16)
    for f in range(NUM_SPARSE_F):
        for g in range(BPW // 16):
            v = plsc.load_gather(
                idxb_v, [g * 16 + lane, jnp.full((16,), f, jnp.int32)])
            idx_v[pl.ds(f * BPW + g * 16, 16)] = v + _OFFSETS[f]

    gets = [
        pltpu.async_copy(table_hbm.at[idx_v.at[pl.ds(f * BPW, BPW)]],
                         rows_v.at[pl.ds(f * BPW, BPW)], gsem)
        for f in range(NUM_SPARSE_F)
    ]

    base = w * ROWS_PW

    def chunk_start(c):
        return jnp.minimum(base + c * CHUNK, VOCAB - CHUNK)

    dmas = [None] * NCHUNK
    dmas[0] = pltpu.async_copy(table_hbm.at[pl.ds(chunk_start(0), CHUNK)],
                               bufs.at[0], nsem)
    accs = tuple(jnp.zeros((16,), jnp.float32) for _ in range(5))

    def accum_chunk(buf, accs):
        def grp(i, accs):
            r = i * 10
            out = []
            for j in range(5):
                a = accs[j]
                v0 = buf[r + 2 * j, :]
                v1 = buf[r + 2 * j + 1, :]
                out.append(a + v0 * v0 + v1 * v1)
            return tuple(out)
        return lax.fori_loop(0, CHUNK // 10, grp, accs)

    for c in range(NCHUNK):
        if c + 1 < NCHUNK:
            dmas[c + 1] = pltpu.async_copy(
                table_hbm.at[pl.ds(chunk_start(c + 1), CHUNK)],
                bufs.at[(c + 1) % 2], nsem)
        dmas[c].wait()
        accs = accum_chunk(bufs.at[c % 2], accs)

    ssq_vec = accs[0] + accs[1] + accs[2] + accs[3] + accs[4]
    v_dup = bufs[(NCHUNK - 1) % 2, 0, :]
    ssq_vec = jnp.where(w == NW - 1, ssq_vec - v_dup * v_dup, ssq_vec)
    ssq_v[...] = ssq_vec
    pltpu.sync_copy(ssq_v, ssq_hbm.at[w])

    @pl.when(w == 0)
    def _cw_ssq():
        acc = cw_v[0, :] * cw_v[0, :]
        for r in range(1, CW_ROWS):
            v = cw_v[r, :]
            acc = acc + v * v
        ssq_v[...] = acc
        pltpu.sync_copy(ssq_v, ssq_hbm.at[NW])

    d16 = jnp.zeros((16,), jnp.float32)
    for f in range(NUM_DENSE_F):
        df = jnp.sum(tab0_v[f, :] * cw_v[f, :])
        d16 = jnp.where(lane == f, df, d16)

    for g in gets:
        g.wait()

    wvecs = [cw_v[NUM_DENSE_F + f, :] for f in range(NUM_SPARSE_F)]

    def body(g, _):
        vec = jnp.zeros((16,), jnp.float32)
        gbase = g * 16
        for j in range(16):
            i = gbase + j
            acc = rd_v[i, :] * d16
            for f in range(NUM_SPARSE_F):
                acc = acc + rows_v[f * BPW + i, :] * wvecs[f]
            vec = jnp.where(lane == j, jnp.sum(acc), vec)
        out_v[pl.ds(gbase, 16)] = vec
        return 0

    lax.fori_loop(0, BPW // 16, body, 0)
    pltpu.sync_copy(out_v, out_hbm.at[pl.ds(w * BPW, BPW)])


def _tc_body(s_ref, cb_ref, ssqm_ref, logits_ref, regs_ref):
    logits_ref[...] = s_ref[...] + cb_ref[...]
    ssqm = ssqm_ref[...]
    t_ssq = jnp.sum(ssqm[:NW, :])
    cw_ssq = jnp.sum(ssqm[NW, :])
    nb = jnp.abs(cb_ref[...])
    regs_ref[...] = REG_COEF * (jnp.sqrt(t_ssq) + jnp.sqrt(cw_ssq) + nb)


_tc_call = pl.pallas_call(
    _tc_body,
    grid=(1,),
    in_specs=[
        pl.BlockSpec((NW, BPW), lambda i: (0, 0)),
        pl.BlockSpec((1, 1), lambda i: (0, 0)),
        pl.BlockSpec((48, 16), lambda i: (0, 0)),
    ],
    out_specs=[
        pl.BlockSpec((NW, BPW), lambda i: (0, 0)),
        pl.BlockSpec((1, 1), lambda i: (0, 0)),
    ],
    out_shape=[
        jax.ShapeDtypeStruct((NW, BPW), jnp.float32),
        jax.ShapeDtypeStruct((1, 1), jnp.float32),
    ],
)


def kernel(raw_dense, raw_sparse, embedding_table, cls_w, cls_b):
    rsp = raw_sparse.astype(jnp.int32)
    cw2 = cls_w.reshape(CW_ROWS, EMB_D)
    rdp = jnp.pad(raw_dense, ((0, 0), (0, EMB_D - NUM_DENSE_F)))
    s, ssqm = _sc_main(embedding_table, rsp, cw2, rdp)

    cb = cls_b.reshape(1, 1)
    logits2, regs = _tc_call(s.reshape(NW, BPW), cb, ssqm)
    return (logits2.reshape(BATCH_N, 1), regs.reshape(()))


# speedup vs baseline: 4.2387x; 1.1624x over previous
import functools

import jax
import jax.numpy as jnp
from jax import lax
from jax.experimental import pallas as pl
from jax.experimental.pallas import tpu as pltpu
from jax.experimental.pallas import tpu_sc as plsc

NUM_DENSE_F = 13
NUM_SPARSE_F = 26
EMB_D = 16
VOCAB = 999999
BATCH_N = 4096
REG_COEF = 1e-05
_OFFSETS = [0] + [13 + j * 38461 for j in range(1, NUM_SPARSE_F)]

NW = 32
BPW = BATCH_N // NW
CW_ROWS = NUM_DENSE_F + NUM_SPARSE_F

CHW = 1024
N_CH = 976
REM_C0 = N_CH * CHW
REM_W = 512
TAIL_V0 = REM_C0 + REM_W
TAIL_N = VOCAB - TAIL_V0
MAXC = 31

_sc_mesh = plsc.VectorSubcoreMesh(core_axis_name="c", subcore_axis_name="s")


@functools.partial(
    pl.kernel,
    mesh=_sc_mesh,
    out_type=(
        jax.ShapeDtypeStruct((VOCAB * EMB_D,), jnp.float32),
        jax.ShapeDtypeStruct((NW * 16,), jnp.float32),
    ),
    scratch_types=[
        pltpu.VMEM((EMB_D, CHW), jnp.float32),
        pltpu.VMEM((EMB_D, CHW), jnp.float32),
        pltpu.VMEM((CHW * EMB_D,), jnp.float32),
        pltpu.VMEM((CHW * EMB_D,), jnp.float32),
        pltpu.VMEM((EMB_D, REM_W), jnp.float32),
        pltpu.VMEM((REM_W * EMB_D,), jnp.float32),
        pltpu.VMEM((TAIL_N * EMB_D,), jnp.float32),
        pltpu.VMEM((16,), jnp.float32),
        pltpu.SemaphoreType.DMA,
        pltpu.SemaphoreType.DMA,
    ],
    compiler_params=pltpu.CompilerParams(
        needs_layout_passes=False, use_tc_tiling_on_sc=True),
)
def _sc_convert(tT_hbm, tail_hbm, out_hbm, ssq_hbm,
                tb0, tb1, ob0, ob1, remb, remo, tailb, acc_ref, isem, osem):
    tb = (tb0, tb1)
    ob = (ob0, ob1)
    w = lax.axis_index("s") * 2 + lax.axis_index("c")
    cnt = jnp.where(w < 16, 31, 30)
    cbase = jnp.where(w < 16, 31 * w, 30 * w + 16)
    lane = lax.iota(jnp.int32, 16)
    acc_ref[...] = jnp.zeros((16,), jnp.float32)

    def col0(k):
        return pl.multiple_of((cbase + k) * CHW, 128)

    def in_copy(k):
        return pltpu.make_async_copy(
            tT_hbm.at[:, pl.ds(col0(k), CHW)], tb[k % 2], isem)

    def out_copy(k):
        return pltpu.make_async_copy(
            ob[k % 2], out_hbm.at[pl.ds(col0(k) * EMB_D, CHW * EMB_D)],
            osem)

    def transpose_chunk(buf, stage, width):
        def grp(i, a):
            for u in range(8):
                vl = i * 8 + u
                row = plsc.load_gather(
                    buf, [lane, jnp.zeros((16,), jnp.int32) + vl])
                stage[pl.ds(vl * EMB_D, EMB_D)] = row
                a = a + row * row
            return a
        return lax.fori_loop(0, width // 8, grp, jnp.zeros((16,), jnp.float32))

    in_copy(0).start()
    for k in range(MAXC):
        if k + 1 < MAXC:
            @pl.when(k + 1 < cnt)
            def _nxt(k=k):
                in_copy(k + 1).start()

        @pl.when(k < cnt)
        def _chunk(k=k):
            in_copy(k).wait()
            if k >= 2:
                out_copy(k - 2).wait()
            a = transpose_chunk(tb[k % 2], ob[k % 2], CHW)
            acc_ref[...] = acc_ref[...] + a
            out_copy(k).start()

    for k in range(MAXC - 3, MAXC):
        @pl.when((k == cnt - 2) | (k == cnt - 1))
        def _drain(k=k):
            out_copy(k).wait()

    @pl.when(w == 31)
    def _rem():
        pltpu.sync_copy(tT_hbm.at[:, pl.ds(REM_C0, REM_W)], remb)
        a = transpose_chunk(remb, remo, REM_W)
        acc_ref[...] = acc_ref[...] + a
        pltpu.sync_copy(remo,
                        out_hbm.at[pl.ds(REM_C0 * EMB_D, REM_W * EMB_D)])

    @pl.when(w == 16)
    def _tail():
        pltpu.sync_copy(tail_hbm, tailb)
        a = jnp.zeros((16,), jnp.float32)
        for r in range(TAIL_N):
            v = tailb[pl.ds(r * EMB_D, EMB_D)]
            a = a + v * v
        acc_ref[...] = acc_ref[...] + a
        pltpu.sync_copy(tailb,
                        out_hbm.at[pl.ds(TAIL_V0 * EMB_D, TAIL_N * EMB_D)])

    ssq_v = acc_ref
    pltpu.sync_copy(ssq_v, ssq_hbm.at[pl.ds(w * 16, 16)])


@functools.partial(
    pl.kernel,
    mesh=_sc_mesh,
    out_type=jax.ShapeDtypeStruct((BATCH_N,), jnp.float32),
    scratch_types=[
        pltpu.VMEM((BPW, NUM_SPARSE_F), jnp.int32),
        pltpu.VMEM((NUM_SPARSE_F * BPW,), jnp.int32),
        pltpu.VMEM((NUM_SPARSE_F * BPW, EMB_D), jnp.float32),
        pltpu.VMEM((CW_ROWS, EMB_D), jnp.float32),
        pltpu.VMEM((BPW, EMB_D), jnp.float32),
        pltpu.VMEM((16, EMB_D), jnp.float32),
        pltpu.VMEM((BPW,), jnp.float32),
        pltpu.SemaphoreType.DMA,
        pltpu.SemaphoreType.DMA,
    ],
    compiler_params=pltpu.CompilerParams(
        needs_layout_passes=False, use_tc_tiling_on_sc=False),
)
def _sc_gather_dot(table_hbm, rsp_hbm, cw_hbm, rd_hbm, out_hbm,
                   idxb_v, idx_v, rows_v, cw_v, rd_v, tab0_v, out_v,
                   gsem, psem):
    w = lax.axis_index("s") * 2 + lax.axis_index("c")
    ins = [
        pltpu.async_copy(rsp_hbm.at[pl.ds(w * BPW, BPW)], idxb_v, psem),
        pltpu.async_copy(cw_hbm, cw_v, psem),
        pltpu.async_copy(rd_hbm.at[pl.ds(w * BPW, BPW)], rd_v, psem),
        pltpu.async_copy(table_hbm.at[pl.ds(0, 16)], tab0_v, psem),
    ]
    for i in ins:
        i.wait()

    lane = lax.iota(jnp.int32, 16)
    for f in range(NUM_SPARSE_F):
        for g in range(BPW // 16):
            v = plsc.load_gather(
                idxb_v, [g * 16 + lane, jnp.full((16,), f, jnp.int32)])
            idx_v[pl.ds(f * BPW + g * 16, 16)] = v + _OFFSETS[f]

    gets = [
        pltpu.async_copy(table_hbm.at[idx_v.at[pl.ds(f * BPW, BPW)]],
                         rows_v.at[pl.ds(f * BPW, BPW)], gsem)
        for f in range(NUM_SPARSE_F)
    ]

    d16 = jnp.zeros((16,), jnp.float32)
    for f in range(NUM_DENSE_F):
        df = jnp.sum(tab0_v[f, :] * cw_v[f, :])
        d16 = jnp.where(lane == f, df, d16)

    for g in gets:
        g.wait()

    wvecs = [cw_v[NUM_DENSE_F + f, :] for f in range(NUM_SPARSE_F)]

    def body(g, _):
        vec = jnp.zeros((16,), jnp.float32)
        gbase = g * 16
        for j in range(16):
            i = gbase + j
            acc = rd_v[i, :] * d16
            for f in range(NUM_SPARSE_F):
                acc = acc + rows_v[f * BPW + i, :] * wvecs[f]
            vec = jnp.where(lane == j, jnp.sum(acc), vec)
        out_v[pl.ds(gbase, 16)] = vec
        return 0

    lax.fori_loop(0, BPW // 16, body, 0)
    pltpu.sync_copy(out_v, out_hbm.at[pl.ds(w * BPW, BPW)])


def _tc_body(s_ref, cb_ref, ssqm_ref, cwp_ref, logits_ref, regs_ref):
    logits_ref[...] = s_ref[...] + cb_ref[...]
    t_ssq = jnp.sum(ssqm_ref[...])
    cwp = cwp_ref[...]
    cw_ssq = jnp.sum(cwp * cwp)
    nb = jnp.abs(cb_ref[...])
    regs_ref[...] = REG_COEF * (jnp.sqrt(t_ssq) + jnp.sqrt(cw_ssq) + nb)


_tc_call = pl.pallas_call(
    _tc_body,
    grid=(1,),
    in_specs=[
        pl.BlockSpec((NW, BPW), lambda i: (0, 0)),
        pl.BlockSpec((1, 1), lambda i: (0, 0)),
        pl.BlockSpec((4, 128), lambda i: (0, 0)),
        pl.BlockSpec((CW_ROWS + 1, EMB_D), lambda i: (0, 0)),
    ],
    out_specs=[
        pl.BlockSpec((NW, BPW), lambda i: (0, 0)),
        pl.BlockSpec((1, 1), lambda i: (0, 0)),
    ],
    out_shape=[
        jax.ShapeDtypeStruct((NW, BPW), jnp.float32),
        jax.ShapeDtypeStruct((1, 1), jnp.float32),
    ],
)


def kernel(raw_dense, raw_sparse, embedding_table, cls_w, cls_b):
    tT = embedding_table.T
    tail = embedding_table[TAIL_V0:, :].reshape(TAIL_N * EMB_D)
    t1d, ssqm = _sc_convert(tT, tail)

    rsp = raw_sparse.astype(jnp.int32)
    cw2 = cls_w.reshape(CW_ROWS, EMB_D)
    rdp = jnp.pad(raw_dense, ((0, 0), (0, EMB_D - NUM_DENSE_F)))
    s = _sc_gather_dot(t1d.reshape(VOCAB, EMB_D), rsp, cw2, rdp)

    cb = cls_b.reshape(1, 1)
    cwp = jnp.pad(cw2, ((0, 1), (0, 0)))
    logits2, regs = _tc_call(s.reshape(NW, BPW), cb,
                             ssqm.reshape(4, 128), cwp)
    return (logits2.reshape(BATCH_N, 1), regs.reshape(()))


# speedup vs baseline: 5.5391x; 1.3068x over previous
import functools

import jax
import jax.numpy as jnp
from jax import lax
from jax.experimental import pallas as pl
from jax.experimental.pallas import tpu as pltpu
from jax.experimental.pallas import tpu_sc as plsc

NUM_DENSE_F = 13
NUM_SPARSE_F = 26
EMB_D = 16
VOCAB = 999999
BATCH_N = 4096
REG_COEF = 1e-05
_OFFSETS = [0] + [13 + j * 38461 for j in range(1, NUM_SPARSE_F)]

NW = 32
BPW = BATCH_N // NW
CW_ROWS = NUM_DENSE_F + NUM_SPARSE_F

TCW = 4096
TC_G = 245
PADV = TC_G * TCW

_sc_mesh = plsc.VectorSubcoreMesh(core_axis_name="c", subcore_axis_name="s")


def _tc_convert_body(tT_ref, cw_ref, cb_ref, out_ref, regs_ref, acc_ref):
    i = pl.program_id(0)
    x = tT_ref[...]
    eye = jnp.eye(EMB_D, dtype=jnp.float32)
    t = lax.dot_general(x, eye, (((0,), (0,)), ((), ())),
                        preferred_element_type=jnp.float32)
    out_ref[...] = jnp.concatenate(
        [t[j * (TCW // 8):(j + 1) * (TCW // 8), :] for j in range(8)],
        axis=1)

    cols = i * TCW + lax.broadcasted_iota(jnp.int32, (EMB_D, TCW), 1)
    xm = jnp.where(cols < VOCAB, x, 0.0)
    g = lax.dot_general(xm, xm, (((1,), (1,)), ((), ())),
                        preferred_element_type=jnp.float32)

    @pl.when(i == 0)
    def _init():
        acc_ref[...] = g

    @pl.when(i > 0)
    def _acc():
        acc_ref[...] = acc_ref[...] + g

    @pl.when(i == TC_G - 1)
    def _fin():
        t_ssq = jnp.sum(acc_ref[...] * eye)
        cw = cw_ref[...]
        cw_ssq = jnp.sum(cw * cw)
        nb = jnp.abs(cb_ref[...])
        regs_ref[...] = REG_COEF * (
            jnp.sqrt(t_ssq) + jnp.sqrt(cw_ssq) + nb)


_tc_convert = pl.pallas_call(
    _tc_convert_body,
    grid=(TC_G,),
    in_specs=[
        pl.BlockSpec((EMB_D, TCW), lambda i: (0, i)),
        pl.BlockSpec((CW_ROWS, EMB_D), lambda i: (0, 0)),
        pl.BlockSpec((1, 1), lambda i: (0, 0)),
    ],
    out_specs=[
        pl.BlockSpec((TCW * EMB_D // 128, 128), lambda i: (i, 0)),
        pl.BlockSpec((1, 1), lambda i: (0, 0)),
    ],
    out_shape=[
        jax.ShapeDtypeStruct((PADV * EMB_D // 128, 128), jnp.float32),
        jax.ShapeDtypeStruct((1, 1), jnp.float32),
    ],
    scratch_shapes=[pltpu.VMEM((EMB_D, EMB_D), jnp.float32)],
)


@functools.partial(
    pl.kernel,
    mesh=_sc_mesh,
    out_type=jax.ShapeDtypeStruct((BATCH_N,), jnp.float32),
    scratch_types=[
        pltpu.VMEM((BPW, NUM_SPARSE_F), jnp.int32),
        pltpu.VMEM((NUM_SPARSE_F * BPW,), jnp.int32),
        pltpu.VMEM((NUM_SPARSE_F * BPW, EMB_D), jnp.float32),
        pltpu.VMEM((CW_ROWS, EMB_D), jnp.float32),
        pltpu.VMEM((BPW, EMB_D), jnp.float32),
        pltpu.VMEM((16, EMB_D), jnp.float32),
        pltpu.VMEM((16,), jnp.int32),
        pltpu.VMEM((16,), jnp.float32),
        pltpu.VMEM((BPW,), jnp.float32),
        pltpu.SemaphoreType.DMA,
        pltpu.SemaphoreType.DMA,
    ],
    compiler_params=pltpu.CompilerParams(
        needs_layout_passes=False, use_tc_tiling_on_sc=False),
)
def _sc_gather_dot(table_hbm, rsp_hbm, cw_hbm, rd_hbm, cb_hbm, out_hbm,
                   idxb_v, idx_v, rows_v, cw_v, rd_v, tab0_v, idx0_v, cb_v,
                   out_v, gsem, psem):
    w = lax.axis_index("s") * 2 + lax.axis_index("c")
    lane = lax.iota(jnp.int32, 16)
    idx0_v[...] = lane * 8
    ins = [
        pltpu.async_copy(rsp_hbm.at[pl.ds(w * BPW, BPW)], idxb_v, psem),
        pltpu.async_copy(cw_hbm, cw_v, psem),
        pltpu.async_copy(rd_hbm.at[pl.ds(w * BPW, BPW)], rd_v, psem),
        pltpu.async_copy(table_hbm.at[idx0_v.at[pl.ds(0, 16)]], tab0_v,
                         psem),
        pltpu.async_copy(cb_hbm, cb_v, psem),
    ]
    for i in ins:
        i.wait()

    for f in range(NUM_SPARSE_F):
        for g in range(BPW // 16):
            v = plsc.load_gather(
                idxb_v, [g * 16 + lane, jnp.full((16,), f, jnp.int32)])
            r = v + _OFFSETS[f]
            s = (r & -4096) + ((r & 511) << 3) + ((r & 4095) >> 9)
            idx_v[pl.ds(f * BPW + g * 16, 16)] = s

    gets = [
        pltpu.async_copy(table_hbm.at[idx_v.at[pl.ds(f * BPW, BPW)]],
                         rows_v.at[pl.ds(f * BPW, BPW)], gsem)
        for f in range(NUM_SPARSE_F)
    ]

    d16 = jnp.zeros((16,), jnp.float32)
    for f in range(NUM_DENSE_F):
        df = jnp.sum(tab0_v[f, :] * cw_v[f, :])
        d16 = jnp.where(lane == f, df, d16)

    for g in gets:
        g.wait()

    wvecs = [cw_v[NUM_DENSE_F + f, :] for f in range(NUM_SPARSE_F)]
    bias = cb_v[...]

    def body(g, _):
        vec = jnp.zeros((16,), jnp.float32)
        gbase = g * 16
        for j in range(16):
            i = gbase + j
            acc = rd_v[i, :] * d16
            for f in range(NUM_SPARSE_F):
                acc = acc + rows_v[f * BPW + i, :] * wvecs[f]
            vec = jnp.where(lane == j, jnp.sum(acc), vec)
        out_v[pl.ds(gbase, 16)] = vec + bias
        return 0

    lax.fori_loop(0, BPW // 16, body, 0)
    pltpu.sync_copy(out_v, out_hbm.at[pl.ds(w * BPW, BPW)])


def kernel(raw_dense, raw_sparse, embedding_table, cls_w, cls_b):
    tT = embedding_table.T
    cw2 = cls_w.reshape(CW_ROWS, EMB_D)
    cb11 = cls_b.reshape(1, 1)
    t2d, regs = _tc_convert(tT, cw2, cb11)

    rsp = raw_sparse.astype(jnp.int32)
    rdp = jnp.pad(raw_dense, ((0, 0), (0, EMB_D - NUM_DENSE_F)))
    cb16 = jnp.broadcast_to(cls_b, (16,))
    logits = _sc_gather_dot(t2d.reshape(PADV, EMB_D), rsp, cw2, rdp, cb16)
    return (logits.reshape(BATCH_N, 1), regs.reshape(()))


# speedup vs baseline: 5.8418x; 1.0546x over previous
import functools

import jax
import jax.numpy as jnp
from jax import lax
from jax.experimental import pallas as pl
from jax.experimental.pallas import tpu as pltpu
from jax.experimental.pallas import tpu_sc as plsc

NUM_DENSE_F = 13
NUM_SPARSE_F = 26
EMB_D = 16
VOCAB = 999999
BATCH_N = 4096
REG_COEF = 1e-05
_OFFSETS = [0] + [13 + j * 38461 for j in range(1, NUM_SPARSE_F)]

NW = 32
BPW = BATCH_N // NW
CW_ROWS = NUM_DENSE_F + NUM_SPARSE_F

TCW = 16384
TC_G = 62
PADV = TC_G * TCW
SLAB = TCW // 8
Q_SH = SLAB.bit_length() - 1

_sc_mesh = plsc.VectorSubcoreMesh(core_axis_name="c", subcore_axis_name="s")


def _tc_convert_body(tT_ref, cw_ref, cb_ref, out_ref, regs_ref, acc_ref):
    i = pl.program_id(0)
    x = tT_ref[...]
    eye = jnp.eye(EMB_D, dtype=jnp.float32)
    t = lax.dot_general(x, eye, (((0,), (0,)), ((), ())),
                        preferred_element_type=jnp.float32)
    out_ref[...] = jnp.concatenate(
        [t[j * SLAB:(j + 1) * SLAB, :] for j in range(8)], axis=1)

    cols = i * TCW + lax.broadcasted_iota(jnp.int32, (EMB_D, TCW), 1)
    xm = jnp.where(cols < VOCAB, x, 0.0)
    g = lax.dot_general(xm, xm, (((1,), (1,)), ((), ())),
                        preferred_element_type=jnp.float32)

    @pl.when(i == 0)
    def _init():
        acc_ref[...] = g

    @pl.when(i > 0)
    def _acc():
        acc_ref[...] = acc_ref[...] + g

    @pl.when(i == TC_G - 1)
    def _fin():
        t_ssq = jnp.sum(acc_ref[...] * eye)
        cw = cw_ref[...]
        cw_ssq = jnp.sum(cw * cw)
        nb = jnp.abs(cb_ref[...])
        regs_ref[...] = REG_COEF * (
            jnp.sqrt(t_ssq) + jnp.sqrt(cw_ssq) + nb)


_tc_convert = pl.pallas_call(
    _tc_convert_body,
    grid=(TC_G,),
    in_specs=[
        pl.BlockSpec((EMB_D, TCW), lambda i: (0, i)),
        pl.BlockSpec((CW_ROWS, EMB_D), lambda i: (0, 0)),
        pl.BlockSpec((1, 1), lambda i: (0, 0)),
    ],
    out_specs=[
        pl.BlockSpec((TCW * EMB_D // 128, 128), lambda i: (i, 0)),
        pl.BlockSpec((1, 1), lambda i: (0, 0)),
    ],
    out_shape=[
        jax.ShapeDtypeStruct((PADV * EMB_D // 128, 128), jnp.float32),
        jax.ShapeDtypeStruct((1, 1), jnp.float32),
    ],
    scratch_shapes=[pltpu.VMEM((EMB_D, EMB_D), jnp.float32)],
)


@functools.partial(
    pl.kernel,
    mesh=_sc_mesh,
    out_type=jax.ShapeDtypeStruct((BATCH_N,), jnp.float32),
    scratch_types=[
        pltpu.VMEM((BPW, NUM_SPARSE_F), jnp.int32),
        pltpu.VMEM((NUM_SPARSE_F * BPW,), jnp.int32),
        pltpu.VMEM((NUM_SPARSE_F * BPW, EMB_D), jnp.float32),
        pltpu.VMEM((CW_ROWS, EMB_D), jnp.float32),
        pltpu.VMEM((BPW, EMB_D), jnp.float32),
        pltpu.VMEM((16, EMB_D), jnp.float32),
        pltpu.VMEM((16,), jnp.int32),
        pltpu.VMEM((16,), jnp.float32),
        pltpu.VMEM((BPW,), jnp.float32),
        pltpu.SemaphoreType.DMA,
        pltpu.SemaphoreType.DMA,
    ],
    compiler_params=pltpu.CompilerParams(
        needs_layout_passes=False, use_tc_tiling_on_sc=False),
)
def _sc_gather_dot(table_hbm, rsp_hbm, cw_hbm, rd_hbm, cb_hbm, out_hbm,
                   idxb_v, idx_v, rows_v, cw_v, rd_v, tab0_v, idx0_v, cb_v,
                   out_v, gsem, psem):
    w = lax.axis_index("s") * 2 + lax.axis_index("c")
    lane = lax.iota(jnp.int32, 16)
    idx0_v[...] = lane * 8
    ins = [
        pltpu.async_copy(rsp_hbm.at[pl.ds(w * BPW, BPW)], idxb_v, psem),
        pltpu.async_copy(cw_hbm, cw_v, psem),
        pltpu.async_copy(rd_hbm.at[pl.ds(w * BPW, BPW)], rd_v, psem),
        pltpu.async_copy(table_hbm.at[idx0_v.at[pl.ds(0, 16)]], tab0_v,
                         psem),
        pltpu.async_copy(cb_hbm, cb_v, psem),
    ]
    for i in ins:
        i.wait()

    for f in range(NUM_SPARSE_F):
        for g in range(BPW // 16):
            v = plsc.load_gather(
                idxb_v, [g * 16 + lane, jnp.full((16,), f, jnp.int32)])
            r = v + _OFFSETS[f]
            s = (r & -TCW) + ((r & (SLAB - 1)) << 3) + ((r & (TCW - 1)) >> Q_SH)
            idx_v[pl.ds(f * BPW + g * 16, 16)] = s

    gets = [
        pltpu.async_copy(table_hbm.at[idx_v.at[pl.ds(f * BPW, BPW)]],
                         rows_v.at[pl.ds(f * BPW, BPW)], gsem)
        for f in range(NUM_SPARSE_F)
    ]

    d16 = jnp.zeros((16,), jnp.float32)
    for f in range(NUM_DENSE_F):
        df = jnp.sum(tab0_v[f, :] * cw_v[f, :])
        d16 = jnp.where(lane == f, df, d16)

    for g in gets:
        g.wait()

    wvecs = [cw_v[NUM_DENSE_F + f, :] for f in range(NUM_SPARSE_F)]
    bias = cb_v[...]

    def body(g, _):
        vec = jnp.zeros((16,), jnp.float32)
        gbase = g * 16
        for j in range(16):
            i = gbase + j
            acc = rd_v[i, :] * d16
            for f in range(NUM_SPARSE_F):
                acc = acc + rows_v[f * BPW + i, :] * wvecs[f]
            vec = jnp.where(lane == j, jnp.sum(acc), vec)
        out_v[pl.ds(gbase, 16)] = vec + bias
        return 0

    lax.fori_loop(0, BPW // 16, body, 0)
    pltpu.sync_copy(out_v, out_hbm.at[pl.ds(w * BPW, BPW)])


def kernel(raw_dense, raw_sparse, embedding_table, cls_w, cls_b):
    tT = embedding_table.T
    cw2 = cls_w.reshape(CW_ROWS, EMB_D)
    cb11 = cls_b.reshape(1, 1)
    t2d, regs = _tc_convert(tT, cw2, cb11)

    rsp = raw_sparse.astype(jnp.int32)
    rdp = jnp.pad(raw_dense, ((0, 0), (0, EMB_D - NUM_DENSE_F)))
    cb16 = jnp.broadcast_to(cls_b, (16,))
    logits = _sc_gather_dot(t2d.reshape(PADV, EMB_D), rsp, cw2, rdp, cb16)
    return (logits.reshape(BATCH_N, 1), regs.reshape(()))


# speedup vs baseline: 5.8863x; 1.0076x over previous
import functools

import jax
import jax.numpy as jnp
from jax import lax
from jax.experimental import pallas as pl
from jax.experimental.pallas import tpu as pltpu
from jax.experimental.pallas import tpu_sc as plsc

NUM_DENSE_F = 13
NUM_SPARSE_F = 26
EMB_D = 16
VOCAB = 999999
BATCH_N = 4096
REG_COEF = 1e-05
_OFFSETS = [0] + [13 + j * 38461 for j in range(1, NUM_SPARSE_F)]

NW = 32
BPW = BATCH_N // NW
CW_ROWS = NUM_DENSE_F + NUM_SPARSE_F

TCW = 32768
TC_G = 31
PADV = TC_G * TCW
SLAB = TCW // 8
Q_SH = SLAB.bit_length() - 1

_sc_mesh = plsc.VectorSubcoreMesh(core_axis_name="c", subcore_axis_name="s")


def _tc_convert_body(tT_ref, cw_ref, cb_ref, out_ref, regs_ref, acc_ref):
    i = pl.program_id(0)
    x = tT_ref[...]
    eye = jnp.eye(EMB_D, dtype=jnp.float32)
    t = lax.dot_general(x, eye, (((0,), (0,)), ((), ())),
                        preferred_element_type=jnp.float32)
    out_ref[...] = jnp.concatenate(
        [t[j * SLAB:(j + 1) * SLAB, :] for j in range(8)], axis=1)

    cols = i * TCW + lax.broadcasted_iota(jnp.int32, (EMB_D, TCW), 1)
    xm = jnp.where(cols < VOCAB, x, 0.0)
    g = lax.dot_general(xm, xm, (((1,), (1,)), ((), ())),
                        preferred_element_type=jnp.float32)

    @pl.when(i == 0)
    def _init():
        acc_ref[...] = g

    @pl.when(i > 0)
    def _acc():
        acc_ref[...] = acc_ref[...] + g

    @pl.when(i == TC_G - 1)
    def _fin():
        t_ssq = jnp.sum(acc_ref[...] * eye)
        cw = cw_ref[...]
        cw_ssq = jnp.sum(cw * cw)
        nb = jnp.abs(cb_ref[...])
        regs_ref[...] = REG_COEF * (
            jnp.sqrt(t_ssq) + jnp.sqrt(cw_ssq) + nb)


_tc_convert = pl.pallas_call(
    _tc_convert_body,
    grid=(TC_G,),
    in_specs=[
        pl.BlockSpec((EMB_D, TCW), lambda i: (0, i)),
        pl.BlockSpec((CW_ROWS, EMB_D), lambda i: (0, 0)),
        pl.BlockSpec((1, 1), lambda i: (0, 0)),
    ],
    out_specs=[
        pl.BlockSpec((TCW * EMB_D // 128, 128), lambda i: (i, 0)),
        pl.BlockSpec((1, 1), lambda i: (0, 0)),
    ],
    out_shape=[
        jax.ShapeDtypeStruct((PADV * EMB_D // 128, 128), jnp.float32),
        jax.ShapeDtypeStruct((1, 1), jnp.float32),
    ],
    scratch_shapes=[pltpu.VMEM((EMB_D, EMB_D), jnp.float32)],
)


@functools.partial(
    pl.kernel,
    mesh=_sc_mesh,
    out_type=jax.ShapeDtypeStruct((BATCH_N,), jnp.float32),
    scratch_types=[
        pltpu.VMEM((BPW, NUM_SPARSE_F), jnp.int32),
        pltpu.VMEM((NUM_SPARSE_F * BPW,), jnp.int32),
        pltpu.VMEM((NUM_SPARSE_F * BPW, EMB_D), jnp.float32),
        pltpu.VMEM((CW_ROWS, EMB_D), jnp.float32),
        pltpu.VMEM((BPW, EMB_D), jnp.float32),
        pltpu.VMEM((16, EMB_D), jnp.float32),
        pltpu.VMEM((16,), jnp.int32),
        pltpu.VMEM((16,), jnp.float32),
        pltpu.VMEM((BPW,), jnp.float32),
        pltpu.SemaphoreType.DMA,
        pltpu.SemaphoreType.DMA,
    ],
    compiler_params=pltpu.CompilerParams(
        needs_layout_passes=False, use_tc_tiling_on_sc=False),
)
def _sc_gather_dot(table_hbm, rsp_hbm, cw_hbm, rd_hbm, cb_hbm, out_hbm,
                   idxb_v, idx_v, rows_v, cw_v, rd_v, tab0_v, idx0_v, cb_v,
                   out_v, gsem, psem):
    w = lax.axis_index("s") * 2 + lax.axis_index("c")
    lane = lax.iota(jnp.int32, 16)
    idx0_v[...] = lane * 8
    ins = [
        pltpu.async_copy(rsp_hbm.at[pl.ds(w * BPW, BPW)], idxb_v, psem),
        pltpu.async_copy(cw_hbm, cw_v, psem),
        pltpu.async_copy(rd_hbm.at[pl.ds(w * BPW, BPW)], rd_v, psem),
        pltpu.async_copy(table_hbm.at[idx0_v.at[pl.ds(0, 16)]], tab0_v,
                         psem),
        pltpu.async_copy(cb_hbm, cb_v, psem),
    ]
    for i in ins:
        i.wait()

    for f in range(NUM_SPARSE_F):
        for g in range(BPW // 16):
            v = plsc.load_gather(
                idxb_v, [g * 16 + lane, jnp.full((16,), f, jnp.int32)])
            r = v + _OFFSETS[f]
            s = (r & -TCW) + ((r & (SLAB - 1)) << 3) + ((r & (TCW - 1)) >> Q_SH)
            idx_v[pl.ds(f * BPW + g * 16, 16)] = s

    gets = [
        pltpu.async_copy(table_hbm.at[idx_v.at[pl.ds(f * BPW, BPW)]],
                         rows_v.at[pl.ds(f * BPW, BPW)], gsem)
        for f in range(NUM_SPARSE_F)
    ]

    d16 = jnp.zeros((16,), jnp.float32)
    for f in range(NUM_DENSE_F):
        df = jnp.sum(tab0_v[f, :] * cw_v[f, :])
        d16 = jnp.where(lane == f, df, d16)

    for g in gets:
        g.wait()

    wvecs = [cw_v[NUM_DENSE_F + f, :] for f in range(NUM_SPARSE_F)]
    bias = cb_v[...]

    def body(g, _):
        vec = jnp.zeros((16,), jnp.float32)
        gbase = g * 16
        for j in range(16):
            i = gbase + j
            acc = rd_v[i, :] * d16
            for f in range(NUM_SPARSE_F):
                acc = acc + rows_v[f * BPW + i, :] * wvecs[f]
            vec = jnp.where(lane == j, jnp.sum(acc), vec)
        out_v[pl.ds(gbase, 16)] = vec + bias
        return 0

    lax.fori_loop(0, BPW // 16, body, 0)
    pltpu.sync_copy(out_v, out_hbm.at[pl.ds(w * BPW, BPW)])


def kernel(raw_dense, raw_sparse, embedding_table, cls_w, cls_b):
    tT = embedding_table.T
    cw2 = cls_w.reshape(CW_ROWS, EMB_D)
    cb11 = cls_b.reshape(1, 1)
    t2d, regs = _tc_convert(tT, cw2, cb11)

    rsp = raw_sparse.astype(jnp.int32)
    rdp = jnp.pad(raw_dense, ((0, 0), (0, EMB_D - NUM_DENSE_F)))
    cb16 = jnp.broadcast_to(cls_b, (16,))
    logits = _sc_gather_dot(t2d.reshape(PADV, EMB_D), rsp, cw2, rdp, cb16)
    return (logits.reshape(BATCH_N, 1), regs.reshape(()))
